# baseline (speedup 1.0000x reference)
import functools
import jax
import jax.numpy as jnp
from jax import lax
from jax.experimental import pallas as pl
from jax.experimental.pallas import tpu as pltpu
from jax.experimental.pallas import tpu_sc as plsc

B, S, H = 2, 4096, 2048
N_IMG, P = 4, 256
NC, NS = 2, 16
NW = NC * NS
RPW = B * S // NW
GRP = NW // (B * N_IMG)
QROWS = P // GRP


def _sc_body(in_hbm, med_hbm, starts_hbm, out_hbm, starts_v):
    c = lax.axis_index("c")
    s = lax.axis_index("s")
    w = c * NS + s
    base = w * RPW
    pltpu.sync_copy(starts_hbm, starts_v)
    pltpu.sync_copy(in_hbm.at[pl.ds(base, RPW)], out_hbm.at[pl.ds(base, RPW)])
    plsc.subcore_barrier()
    k = w // GRP
    q = w % GRP
    v = starts_v[...]
    lane = lax.broadcasted_iota(jnp.int32, (16,), 0)
    start_k = jnp.max(jnp.where(lane == k, v, 0))
    dst = start_k + q * QROWS
    src = k * P + q * QROWS
    pltpu.sync_copy(med_hbm.at[pl.ds(src, QROWS)], out_hbm.at[pl.ds(dst, QROWS)])


@functools.partial(jax.jit, static_argnums=())
def _sc_call(in2, med2, g16):
    mesh = plsc.VectorSubcoreMesh(
        core_axis_name="c", subcore_axis_name="s", num_cores=NC, num_subcores=NS
    )
    return pl.kernel(
        _sc_body,
        out_type=jax.ShapeDtypeStruct((B * S, H), jnp.float32),
        mesh=mesh,
        scratch_types=[pltpu.VMEM((16,), jnp.int32)],
        compiler_params=pltpu.CompilerParams(
            use_tc_tiling_on_sc=False, needs_layout_passes=False
        ),
    )(in2, med2, g16)


def kernel(inputs_embeds, media_features, media_start_positions):
    in2 = inputs_embeds.reshape(B * S, H)
    med2 = media_features.reshape(B * N_IMG * P, H)
    g = (
        media_start_positions.astype(jnp.int32)
        + jnp.arange(B, dtype=jnp.int32)[:, None] * S
    ).reshape(-1)
    g16 = jnp.pad(g, (0, 16 - B * N_IMG))
    out = _sc_call(in2, med2, g16)
    return out.reshape(B, S, H)
